# Initial kernel scaffold; baseline (speedup 1.0000x reference)
import jax
import jax.numpy as jnp
from jax.experimental import pallas as pl


def kernel(x, edge_list, edge_attr, W, b, W_fc, b_fc):
    raise NotImplementedError("write your pallas kernel here")


# speedup vs baseline: 11.8846x; 11.8846x over previous
import functools

import jax
import jax.numpy as jnp
from jax import lax
from jax.experimental import pallas as pl
from jax.experimental.pallas import tpu as pltpu
from jax.experimental.pallas import tpu_sc as plsc

N = 10000
E = 320000
F_IN = 128
H = 64

NC = 2
NS = 16
NW = NC * NS
EPW = E // NW
CH = 80
NCHUNK = EPW // CH

DEG_PAD = 10240
DEG_SL = DEG_PAD // NS
ROWS_PAD = 10240
ROWS_SL = ROWS_PAD // NS

_mesh = plsc.VectorSubcoreMesh(core_axis_name="c", subcore_axis_name="s")


@functools.partial(
    pl.kernel,
    mesh=_mesh,
    out_type=jax.ShapeDtypeStruct((NC, DEG_PAD), jnp.float32),
    scratch_types=[
        pltpu.VMEM((CH,), jnp.int32),
        pltpu.VMEM((CH,), jnp.float32),
        pltpu.VMEM((DEG_SL,), jnp.float32),
        pltpu.VMEM_SHARED((DEG_PAD,), jnp.float32),
    ],
    compiler_params=pltpu.CompilerParams(use_tc_tiling_on_sc=False),
)
def _deg_sc(dst_hbm, ew_hbm, out_hbm, dst_v, ew_v, zv, deg_sh):
    cid = lax.axis_index("c")
    sid = lax.axis_index("s")
    wid = sid * NC + cid

    def zbody(k, c):
        zv[pl.ds(k * 16, 16)] = jnp.zeros((16,), jnp.float32)
        return c
    lax.fori_loop(0, DEG_SL // 16, zbody, 0)
    pltpu.sync_copy(zv, deg_sh.at[pl.ds(sid * DEG_SL, DEG_SL)])
    plsc.subcore_barrier()

    def chunk(j, c):
        base = wid * EPW + j * CH
        pltpu.sync_copy(dst_hbm.at[pl.ds(base, CH)], dst_v)
        pltpu.sync_copy(ew_hbm.at[pl.ds(base, CH)], ew_v)
        pltpu.sync_copy(ew_v, deg_sh.at[dst_v], add=True)
        return c
    lax.fori_loop(0, NCHUNK, chunk, 0)
    plsc.subcore_barrier()

    pltpu.sync_copy(deg_sh.at[pl.ds(sid * DEG_SL, DEG_SL)],
                    out_hbm.at[cid, pl.ds(sid * DEG_SL, DEG_SL)])


@functools.partial(
    pl.kernel,
    mesh=_mesh,
    out_type=jax.ShapeDtypeStruct((NC, ROWS_PAD, H), jnp.float32),
    scratch_types=[
        pltpu.VMEM((CH,), jnp.int32),
        pltpu.VMEM((CH,), jnp.int32),
        pltpu.VMEM((CH,), jnp.float32),
        pltpu.VMEM((CH, H), jnp.float32),
        pltpu.VMEM((ROWS_SL, H), jnp.float32),
        pltpu.VMEM_SHARED((ROWS_PAD, H), jnp.float32),
        pltpu.SemaphoreType.DMA,
    ],
    compiler_params=pltpu.CompilerParams(use_tc_tiling_on_sc=False),
)
def _agg_sc(src_hbm, dst_hbm, ew_hbm, hs_hbm, out_hbm,
            src_v, dst_v, ew_v, rows_v, zv, g_sh, sem):
    cid = lax.axis_index("c")
    sid = lax.axis_index("s")
    wid = sid * NC + cid

    def zbody(r, c):
        for f in range(H // 16):
            zv[r, pl.ds(f * 16, 16)] = jnp.zeros((16,), jnp.float32)
        return c
    lax.fori_loop(0, ROWS_SL, zbody, 0)
    pltpu.sync_copy(zv, g_sh.at[pl.ds(sid * ROWS_SL, ROWS_SL)])
    plsc.subcore_barrier()

    def chunk(j, c):
        base = wid * EPW + j * CH
        pltpu.sync_copy(src_hbm.at[pl.ds(base, CH)], src_v)
        pltpu.sync_copy(dst_hbm.at[pl.ds(base, CH)], dst_v)
        pltpu.sync_copy(ew_hbm.at[pl.ds(base, CH)], ew_v)
        pltpu.async_copy(hs_hbm.at[src_v], rows_v, sem).wait()

        def scale(g, cc):
            ew16 = ew_v[pl.ds(g * 16, 16)]
            for l in range(16):
                s = ew16[l]
                r = g * 16 + l
                for f in range(H // 16):
                    rows_v[r, pl.ds(f * 16, 16)] = (
                        rows_v[r, pl.ds(f * 16, 16)] * s)
            return cc
        lax.fori_loop(0, CH // 16, scale, 0)
        pltpu.sync_copy(rows_v, g_sh.at[dst_v], add=True)
        return c
    lax.fori_loop(0, NCHUNK, chunk, 0)
    plsc.subcore_barrier()

    pltpu.sync_copy(g_sh.at[pl.ds(sid * ROWS_SL, ROWS_SL)],
                    out_hbm.at[cid, pl.ds(sid * ROWS_SL, ROWS_SL)])


_RB = 1000


def _t1_body(deg_ref, x_ref, w_ref, hs_ref):
    deg = deg_ref[0] + deg_ref[1] + 1.0
    dis = lax.rsqrt(deg)
    h = jnp.dot(x_ref[...], w_ref[...], preferred_element_type=jnp.float32)
    hs_ref[...] = dis * h


def _t1(deg3, x, W):
    return pl.pallas_call(
        _t1_body,
        grid=(N // _RB,),
        in_specs=[
            pl.BlockSpec((NC, _RB, 1), lambda i: (0, i, 0)),
            pl.BlockSpec((_RB, F_IN), lambda i: (i, 0)),
            pl.BlockSpec((F_IN, H), lambda i: (0, 0)),
        ],
        out_specs=pl.BlockSpec((_RB, H), lambda i: (i, 0)),
        out_shape=jax.ShapeDtypeStruct((N, H), jnp.float32),
    )(deg3, x, W)


def _t2_body(deg_ref, g_ref, hs_ref, b_ref, wfc_ref, bfc_ref, out_ref):
    i = pl.program_id(0)

    @pl.when(i == 0)
    def _():
        out_ref[...] = jnp.zeros_like(out_ref)

    deg = deg_ref[0] + deg_ref[1] + 1.0
    dis = lax.rsqrt(deg)
    gsum = g_ref[0] + g_ref[1]
    o = jnp.maximum(dis * (gsum + hs_ref[...]) + b_ref[...], 0.0)
    out_ref[...] = out_ref[...] + jnp.sum(o * wfc_ref[...])

    @pl.when(i == pl.num_programs(0) - 1)
    def _():
        acc = out_ref[...] + bfc_ref[...]
        out_ref[...] = 1.0 / (1.0 + jnp.exp(-acc))


def _t2(deg3, g2, hs, b2, wfc2, bfc2):
    return pl.pallas_call(
        _t2_body,
        grid=(N // _RB,),
        in_specs=[
            pl.BlockSpec((NC, _RB, 1), lambda i: (0, i, 0)),
            pl.BlockSpec((NC, _RB, H), lambda i: (0, i, 0)),
            pl.BlockSpec((_RB, H), lambda i: (i, 0)),
            pl.BlockSpec((1, H), lambda i: (0, 0)),
            pl.BlockSpec((_RB, H), lambda i: (i, 0)),
            pl.BlockSpec((1, 1), lambda i: (0, 0)),
        ],
        out_specs=pl.BlockSpec((1, 1), lambda i: (0, 0)),
        out_shape=jax.ShapeDtypeStruct((1, 1), jnp.float32),
    )(deg3, g2, hs, b2, wfc2, bfc2)


def kernel(x, edge_list, edge_attr, W, b, W_fc, b_fc):
    src = edge_list[0]
    dst = edge_list[1]
    deg_raw = _deg_sc(dst, edge_attr)
    deg3 = deg_raw[:, :N].reshape(NC, N, 1)
    hs = _t1(deg3, x, W)
    g2 = _agg_sc(src, dst, edge_attr, hs)[:, :N]
    y = _t2(deg3, g2, hs, b.reshape(1, H),
            W_fc.reshape(N, H), b_fc.reshape(1, 1))
    return y.reshape(())


# speedup vs baseline: 14.7353x; 1.2399x over previous
import functools

import jax
import jax.numpy as jnp
from jax import lax
from jax.experimental import pallas as pl
from jax.experimental.pallas import tpu as pltpu
from jax.experimental.pallas import tpu_sc as plsc

N = 10000
E = 320000
F_IN = 128
H = 64

NC = 2
NS = 16
NW = NC * NS
EPW = E // NW
CH = 80
NCHUNK = EPW // CH

ROWS_PAD = 10240
ROWS_SL = ROWS_PAD // NS

_mesh = plsc.VectorSubcoreMesh(core_axis_name="c", subcore_axis_name="s")
_sc_params = pltpu.CompilerParams(use_tc_tiling_on_sc=False,
                                  needs_layout_passes=False)


@functools.partial(
    pl.kernel,
    mesh=_mesh,
    out_type=jax.ShapeDtypeStruct((NW, N), jnp.float32),
    scratch_types=[
        pltpu.VMEM((NCHUNK, CH), jnp.int32),
        pltpu.VMEM((NCHUNK, CH), jnp.float32),
        pltpu.VMEM((N,), jnp.float32),
    ],
    compiler_params=_sc_params,
)
def _deg_sc(dst_hbm, ew_hbm, out_hbm, dst_t, ew_t, dloc):
    cid = lax.axis_index("c")
    sid = lax.axis_index("s")
    wid = sid * NC + cid

    def zb(k, c):
        dloc[pl.ds(k * 16, 16)] = jnp.zeros((16,), jnp.float32)
        return c
    lax.fori_loop(0, N // 16, zb, 0)

    pltpu.sync_copy(dst_hbm.at[wid], dst_t)
    pltpu.sync_copy(ew_hbm.at[wid], ew_t)

    def chunk(j, c):
        for g in range(CH // 16):
            d16 = dst_t[j, pl.ds(g * 16, 16)]
            w16 = ew_t[j, pl.ds(g * 16, 16)]
            plsc.addupdate_scatter(dloc, [d16], w16)
        return c
    lax.fori_loop(0, NCHUNK, chunk, 0)

    pltpu.sync_copy(dloc, out_hbm.at[wid])


@functools.partial(
    pl.kernel,
    mesh=_mesh,
    out_type=jax.ShapeDtypeStruct((NC, ROWS_PAD, H), jnp.float32),
    scratch_types=[
        pltpu.VMEM((NCHUNK, CH), jnp.int32),
        pltpu.VMEM((NCHUNK, CH), jnp.int32),
        pltpu.VMEM((NCHUNK, CH), jnp.float32),
        pltpu.VMEM((CH, H), jnp.float32),
        pltpu.VMEM((CH, H), jnp.float32),
        pltpu.VMEM((ROWS_SL, H), jnp.float32),
        pltpu.VMEM_SHARED((ROWS_PAD, H), jnp.float32),
        pltpu.SemaphoreType.DMA,
        pltpu.SemaphoreType.DMA,
    ],
    compiler_params=_sc_params,
)
def _agg_sc(src_hbm, dst_hbm, ew_hbm, hs_hbm, out_hbm,
            src_t, dst_t, ew_t, rows0, rows1, zv, g_sh, sem0, sem1):
    cid = lax.axis_index("c")
    sid = lax.axis_index("s")
    wid = sid * NC + cid
    rows = (rows0, rows1)
    sems = (sem0, sem1)

    def zb(r, c):
        for f in range(H // 16):
            zv[r, pl.ds(f * 16, 16)] = jnp.zeros((16,), jnp.float32)
        return c
    lax.fori_loop(0, ROWS_SL, zb, 0)
    pltpu.sync_copy(zv, g_sh.at[pl.ds(sid * ROWS_SL, ROWS_SL)])

    pltpu.sync_copy(src_hbm.at[wid], src_t)
    pltpu.sync_copy(dst_hbm.at[wid], dst_t)
    pltpu.sync_copy(ew_hbm.at[wid], ew_t)
    plsc.subcore_barrier()

    def issue(j, b):
        pltpu.async_copy(hs_hbm.at[src_t.at[j]], rows[b], sems[b])

    def process(j, b):
        pltpu.make_async_copy(hs_hbm.at[pl.ds(0, CH)], rows[b], sems[b]).wait()

        def scale(g, cc):
            ew16 = ew_t[j, pl.ds(g * 16, 16)]
            for l in range(16):
                s = ew16[l]
                r = g * 16 + l
                for f in range(H // 16):
                    rows[b][r, pl.ds(f * 16, 16)] = (
                        rows[b][r, pl.ds(f * 16, 16)] * s)
            return cc
        lax.fori_loop(0, CH // 16, scale, 0)
        pltpu.sync_copy(rows[b], g_sh.at[dst_t.at[j]], add=True)

    issue(0, 0)

    def pair(p, c):
        j = p * 2
        issue(j + 1, 1)
        process(j, 0)

        @pl.when(j + 2 < NCHUNK)
        def _():
            issue(j + 2, 0)
        process(j + 1, 1)
        return c
    lax.fori_loop(0, NCHUNK // 2, pair, 0)
    process(NCHUNK - 1, 0)

    plsc.subcore_barrier()
    pltpu.sync_copy(g_sh.at[pl.ds(sid * ROWS_SL, ROWS_SL)],
                    out_hbm.at[cid, pl.ds(sid * ROWS_SL, ROWS_SL)])


_RB = 1000


def _t1_body(deg_ref, x_ref, w_ref, hs_ref):
    deg = jnp.sum(deg_ref[...], axis=0) + 1.0
    dis = lax.rsqrt(deg)
    h = jnp.dot(x_ref[...], w_ref[...], preferred_element_type=jnp.float32)
    hs_ref[...] = dis * h


def _t1(deg3, x, W):
    return pl.pallas_call(
        _t1_body,
        grid=(N // _RB,),
        in_specs=[
            pl.BlockSpec((NW, _RB, 1), lambda i: (0, i, 0)),
            pl.BlockSpec((_RB, F_IN), lambda i: (i, 0)),
            pl.BlockSpec((F_IN, H), lambda i: (0, 0)),
        ],
        out_specs=pl.BlockSpec((_RB, H), lambda i: (i, 0)),
        out_shape=jax.ShapeDtypeStruct((N, H), jnp.float32),
    )(deg3, x, W)


def _t2_body(deg_ref, g_ref, hs_ref, b_ref, wfc_ref, bfc_ref, out_ref):
    i = pl.program_id(0)

    @pl.when(i == 0)
    def _():
        out_ref[...] = jnp.zeros_like(out_ref)

    deg = jnp.sum(deg_ref[...], axis=0) + 1.0
    dis = lax.rsqrt(deg)
    gsum = g_ref[0] + g_ref[1]
    o = jnp.maximum(dis * (gsum + hs_ref[...]) + b_ref[...], 0.0)
    out_ref[...] = out_ref[...] + jnp.sum(o * wfc_ref[...])

    @pl.when(i == pl.num_programs(0) - 1)
    def _():
        acc = out_ref[...] + bfc_ref[...]
        out_ref[...] = 1.0 / (1.0 + jnp.exp(-acc))


def _t2(deg3, g2, hs, b2, wfc2, bfc2):
    return pl.pallas_call(
        _t2_body,
        grid=(N // _RB,),
        in_specs=[
            pl.BlockSpec((NW, _RB, 1), lambda i: (0, i, 0)),
            pl.BlockSpec((NC, _RB, H), lambda i: (0, i, 0)),
            pl.BlockSpec((_RB, H), lambda i: (i, 0)),
            pl.BlockSpec((1, H), lambda i: (0, 0)),
            pl.BlockSpec((_RB, H), lambda i: (i, 0)),
            pl.BlockSpec((1, 1), lambda i: (0, 0)),
        ],
        out_specs=pl.BlockSpec((1, 1), lambda i: (0, 0)),
        out_shape=jax.ShapeDtypeStruct((1, 1), jnp.float32),
    )(deg3, g2, hs, b2, wfc2, bfc2)


def kernel(x, edge_list, edge_attr, W, b, W_fc, b_fc):
    src3 = edge_list[0].reshape(NW, NCHUNK, CH)
    dst3 = edge_list[1].reshape(NW, NCHUNK, CH)
    ew3 = edge_attr.reshape(NW, NCHUNK, CH)
    deg_raw = _deg_sc(dst3, ew3)
    deg3 = deg_raw.reshape(NW, N, 1)
    hs = _t1(deg3, x, W)
    g2 = _agg_sc(src3, dst3, ew3, hs)[:, :N]
    y = _t2(deg3, g2, hs, b.reshape(1, H),
            W_fc.reshape(N, H), b_fc.reshape(1, 1))
    return y.reshape(())
